# Initial kernel scaffold; baseline (speedup 1.0000x reference)
import jax
import jax.numpy as jnp
from jax.experimental import pallas as pl


def kernel(pos, edge_index, a, W1, b1, W2, b2, W3, b3):
    raise NotImplementedError("write your pallas kernel here")


# speedup vs baseline: 24.7443x; 24.7443x over previous
import functools

import jax
import jax.numpy as jnp
from jax import lax
from jax.experimental import pallas as pl
from jax.experimental.pallas import tpu as pltpu
from jax.experimental.pallas import tpu_sc as plsc

MAX_R = 0.1
YNORM = 1.0

_SC_MESH = plsc.VectorSubcoreMesh(core_axis_name="c", subcore_axis_name="s")
_SC_PARAMS = pltpu.CompilerParams(needs_layout_passes=False)


def _make_gather(N, E):
    n_shards = 8
    shard = E // n_shards
    C = 2000
    nch = shard // C
    assert shard % C == 0 and C % 16 == 0

    ev = jax.ShapeDtypeStruct((E,), jnp.float32)

    @functools.partial(
        pl.kernel,
        out_type=(ev, ev, ev, ev),
        mesh=_SC_MESH,
        compiler_params=_SC_PARAMS,
        scratch_types=[
            pltpu.VMEM((N,), jnp.float32),
            pltpu.VMEM((C,), jnp.int32),
            pltpu.VMEM((C,), jnp.int32),
            pltpu.VMEM((C,), jnp.float32),
        ],
    )
    def gather_kernel(px, py, a0h, a1h, src_hbm, dst_hbm,
                      odx, ody, oa0, oa1, tab_v, ids_v, idd_v, out_v):
        wid = lax.axis_index("c") * 16 + lax.axis_index("s")
        col = wid % 4
        sid = wid // 4
        tabs = [px, py, a0h, a1h]
        outs = [odx, ody, oa0, oa1]

        for k in range(4):
            @pl.when(col == k)
            def _(k=k):
                pltpu.sync_copy(tabs[k], tab_v)

                @pl.loop(0, nch)
                def _(ci):
                    base = sid * shard + ci * C
                    pltpu.sync_copy(dst_hbm.at[pl.ds(base, C)], idd_v)
                    if k < 2:
                        pltpu.sync_copy(src_hbm.at[pl.ds(base, C)], ids_v)

                        @pl.loop(0, C // 16, unroll=8)
                        def _(g):
                            o = g * 16
                            vs = plsc.load_gather(tab_v, [ids_v[pl.ds(o, 16)]])
                            vd = plsc.load_gather(tab_v, [idd_v[pl.ds(o, 16)]])
                            out_v[pl.ds(o, 16)] = (vs - vd) * (1.0 / MAX_R)
                    else:
                        @pl.loop(0, C // 16, unroll=8)
                        def _(g):
                            o = g * 16
                            out_v[pl.ds(o, 16)] = plsc.load_gather(
                                tab_v, [idd_v[pl.ds(o, 16)]])
                    pltpu.sync_copy(out_v, outs[k].at[pl.ds(base, C)])

    return gather_kernel


def _make_mlp(E):
    BLK = 2048
    assert E % BLK == 0

    def mlp_body(dx_ref, dy_ref, a0_ref, a1_ref,
                 w1_ref, b1_ref, w2_ref, b2_ref, w3_ref, b3_ref,
                 ox_ref, oy_ref):
        d0 = dx_ref[...].reshape(1, BLK)
        d1 = dy_ref[...].reshape(1, BLK)
        r = jnp.sqrt(d0 * d0 + d1 * d1)
        feat = jnp.concatenate(
            [d0, d1, r,
             a0_ref[...].reshape(1, BLK), a1_ref[...].reshape(1, BLK),
             jnp.zeros((3, BLK), jnp.float32)], axis=0)
        dn = (((1,), (0,)), ((), ()))
        h = lax.dot_general(w1_ref[...], feat, dn,
                            preferred_element_type=jnp.float32) + b1_ref[...]
        h = jnp.maximum(h, 0.0)
        h = lax.dot_general(w2_ref[...], h, dn,
                            preferred_element_type=jnp.float32) + b2_ref[...]
        h = jnp.maximum(h, 0.0)
        o = lax.dot_general(w3_ref[...], h, dn,
                            preferred_element_type=jnp.float32) + b3_ref[...]
        ox_ref[...] = o[0, :]
        oy_ref[...] = o[1, :]

    ev = jax.ShapeDtypeStruct((E,), jnp.float32)
    edge_spec = pl.BlockSpec((BLK,), lambda i: (i,))
    full = lambda s: pl.BlockSpec(s, lambda i: (0, 0))
    return pl.pallas_call(
        mlp_body,
        grid=(E // BLK,),
        in_specs=[edge_spec, edge_spec, edge_spec, edge_spec,
                  full((64, 8)), full((64, 1)), full((64, 64)),
                  full((64, 1)), full((8, 64)), full((8, 1))],
        out_specs=(edge_spec, edge_spec),
        out_shape=(ev, ev),
    )


def _make_scatter(N, E):
    shard = E // 16
    C = 2000
    nch = shard // C
    assert shard % C == 0 and C % 16 == 0
    SL = -(-N // 16)
    SL = (SL + 7) // 8 * 8
    NPAD = SL * 16

    @functools.partial(
        pl.kernel,
        out_type=(jax.ShapeDtypeStruct((32 * NPAD,), jnp.float32),
                  jax.ShapeDtypeStruct((2 * NPAD,), jnp.float32)),
        mesh=_SC_MESH,
        compiler_params=_SC_PARAMS,
        scratch_types=[
            pltpu.VMEM((NPAD,), jnp.float32),
            pltpu.VMEM((C,), jnp.int32),
            pltpu.VMEM((C,), jnp.float32),
            pltpu.VMEM((SL,), jnp.float32),
            pltpu.VMEM((SL,), jnp.float32),
        ],
    )
    def scatter_kernel(mx_hbm, my_hbm, dst_hbm, zero_hbm, part_hbm, out_hbm,
                       acc_v, dst_v, msg_v, red_v, tmp_v):
        c = lax.axis_index("c")
        sid = lax.axis_index("s")
        pltpu.sync_copy(zero_hbm, acc_v)

        @pl.loop(0, nch)
        def _(ci):
            base = sid * shard + ci * C
            pltpu.sync_copy(dst_hbm.at[pl.ds(base, C)], dst_v)

            @pl.when(c == 0)
            def _():
                pltpu.sync_copy(mx_hbm.at[pl.ds(base, C)], msg_v)

            @pl.when(c == 1)
            def _():
                pltpu.sync_copy(my_hbm.at[pl.ds(base, C)], msg_v)

            @pl.loop(0, C // 16, unroll=8)
            def _(g):
                o = g * 16
                plsc.addupdate_scatter(
                    acc_v, [dst_v[pl.ds(o, 16)]], msg_v[pl.ds(o, 16)])

        wid = c * 16 + sid
        pltpu.sync_copy(acc_v, part_hbm.at[pl.ds(wid * NPAD, NPAD)])
        plsc.subcore_barrier()

        pltpu.sync_copy(part_hbm.at[pl.ds(c * 16 * NPAD + sid * SL, SL)],
                        red_v)

        @pl.loop(1, 16)
        def _(p):
            pltpu.sync_copy(
                part_hbm.at[pl.ds((c * 16 + p) * NPAD + sid * SL, SL)], tmp_v)

            @pl.loop(0, SL // 16, unroll=8)
            def _(g):
                o = g * 16
                red_v[pl.ds(o, 16)] = red_v[pl.ds(o, 16)] + tmp_v[pl.ds(o, 16)]

        pltpu.sync_copy(red_v, out_hbm.at[pl.ds(c * NPAD + sid * SL, SL)])

    return scatter_kernel, NPAD


def kernel(pos, edge_index, a, W1, b1, W2, b2, W3, b3):
    N = pos.shape[0]
    E = edge_index.shape[1]

    px = pos[:, 0]
    py = pos[:, 1]
    a0 = a[:, 0]
    a1 = a[:, 1]
    src = edge_index[0]
    dst = edge_index[1]

    w1p = jnp.concatenate(
        [W1.T, jnp.zeros((W1.shape[1], 3), jnp.float32)], axis=1)
    b1c = b1.reshape(-1, 1)
    w2t = W2.T
    b2c = b2.reshape(-1, 1)
    w3p = jnp.concatenate(
        [W3.T, jnp.zeros((6, W3.shape[0]), jnp.float32)], axis=0)
    b3c = jnp.concatenate([b3, jnp.zeros((6,), jnp.float32)]).reshape(-1, 1)

    dxs, dys, a0g, a1g = _make_gather(N, E)(px, py, a0, a1, src, dst)
    mx, my = _make_mlp(E)(dxs, dys, a0g, a1g, w1p, b1c, w2t, b2c, w3p, b3c)

    scatter, NPAD = _make_scatter(N, E)
    zeros_n = jnp.zeros((NPAD,), jnp.float32)
    _, accflat = scatter(mx, my, dst, zeros_n)
    return accflat.reshape(2, NPAD)[:, :N].T * YNORM


# speedup vs baseline: 43.3341x; 1.7513x over previous
import functools

import jax
import jax.numpy as jnp
from jax import lax
from jax.experimental import pallas as pl
from jax.experimental.pallas import tpu as pltpu
from jax.experimental.pallas import tpu_sc as plsc

MAX_R = 0.1
YNORM = 1.0

_SC_MESH = plsc.VectorSubcoreMesh(core_axis_name="c", subcore_axis_name="s")
_SC_PARAMS = pltpu.CompilerParams(needs_layout_passes=False)


def _make_gather(N, E):
    n_shards = 8
    shard = E // n_shards
    C = 8000
    nch = shard // C
    assert shard % C == 0 and C % 16 == 0

    ev = jax.ShapeDtypeStruct((E,), jnp.float32)

    @functools.partial(
        pl.kernel,
        out_type=(ev, ev, ev, ev),
        mesh=_SC_MESH,
        compiler_params=_SC_PARAMS,
        scratch_types=[
            pltpu.VMEM((N,), jnp.float32),
            pltpu.VMEM((C,), jnp.int32),
            pltpu.VMEM((C,), jnp.int32),
            pltpu.VMEM((C,), jnp.float32),
        ],
    )
    def gather_kernel(px, py, a0h, a1h, src_hbm, dst_hbm,
                      odx, ody, oa0, oa1, tab_v, ids_v, idd_v, out_v):
        wid = lax.axis_index("c") * 16 + lax.axis_index("s")
        col = wid % 4
        sid = wid // 4
        tabs = [px, py, a0h, a1h]
        outs = [odx, ody, oa0, oa1]

        for k in range(4):
            @pl.when(col == k)
            def _(k=k):
                pltpu.sync_copy(tabs[k], tab_v)

                @pl.loop(0, nch)
                def _(ci):
                    base = sid * shard + ci * C
                    pltpu.sync_copy(dst_hbm.at[pl.ds(base, C)], idd_v)
                    if k < 2:
                        pltpu.sync_copy(src_hbm.at[pl.ds(base, C)], ids_v)

                        @pl.loop(0, C // 16, unroll=8)
                        def _(g):
                            o = g * 16
                            vs = plsc.load_gather(tab_v, [ids_v[pl.ds(o, 16)]])
                            vd = plsc.load_gather(tab_v, [idd_v[pl.ds(o, 16)]])
                            out_v[pl.ds(o, 16)] = (vs - vd) * (1.0 / MAX_R)
                    else:
                        @pl.loop(0, C // 16, unroll=8)
                        def _(g):
                            o = g * 16
                            out_v[pl.ds(o, 16)] = plsc.load_gather(
                                tab_v, [idd_v[pl.ds(o, 16)]])
                    pltpu.sync_copy(out_v, outs[k].at[pl.ds(base, C)])

    return gather_kernel


def _make_mlp(E):
    BLK = 5120
    assert E % BLK == 0

    def mlp_body(dx_ref, dy_ref, a0_ref, a1_ref,
                 w1_ref, b1_ref, w2_ref, b2_ref, w3_ref, b3_ref,
                 ox_ref, oy_ref):
        d0 = dx_ref[...].reshape(1, BLK)
        d1 = dy_ref[...].reshape(1, BLK)
        r = jnp.sqrt(d0 * d0 + d1 * d1)
        feat = jnp.concatenate(
            [d0, d1, r,
             a0_ref[...].reshape(1, BLK), a1_ref[...].reshape(1, BLK),
             jnp.zeros((3, BLK), jnp.float32)], axis=0)
        dn = (((1,), (0,)), ((), ()))
        h = lax.dot_general(w1_ref[...], feat.astype(jnp.bfloat16), dn,
                            preferred_element_type=jnp.float32) + b1_ref[...]
        h = jnp.maximum(h, 0.0).astype(jnp.bfloat16)
        h = lax.dot_general(w2_ref[...], h, dn,
                            preferred_element_type=jnp.float32) + b2_ref[...]
        h = jnp.maximum(h, 0.0).astype(jnp.bfloat16)
        o = lax.dot_general(w3_ref[...], h, dn,
                            preferred_element_type=jnp.float32) + b3_ref[...]
        ox_ref[...] = o[0, :]
        oy_ref[...] = o[1, :]

    ev = jax.ShapeDtypeStruct((E,), jnp.float32)
    edge_spec = pl.BlockSpec((BLK,), lambda i: (i,))
    full = lambda s: pl.BlockSpec(s, lambda i: (0, 0))
    return pl.pallas_call(
        mlp_body,
        grid=(E // BLK,),
        in_specs=[edge_spec, edge_spec, edge_spec, edge_spec,
                  full((64, 8)), full((64, 1)), full((64, 64)),
                  full((64, 1)), full((8, 64)), full((8, 1))],
        out_specs=(edge_spec, edge_spec),
        out_shape=(ev, ev),
    )


def _make_scatter(N, E):
    shard = E // 16
    C = 8000
    nch = shard // C
    assert shard % C == 0 and C % 16 == 0
    SL = -(-N // 16)
    SL = (SL + 7) // 8 * 8
    NPAD = SL * 16

    @functools.partial(
        pl.kernel,
        out_type=(jax.ShapeDtypeStruct((32 * NPAD,), jnp.float32),
                  jax.ShapeDtypeStruct((2 * NPAD,), jnp.float32)),
        mesh=_SC_MESH,
        compiler_params=_SC_PARAMS,
        scratch_types=[
            pltpu.VMEM((NPAD,), jnp.float32),
            pltpu.VMEM((C,), jnp.int32),
            pltpu.VMEM((C,), jnp.float32),
            pltpu.VMEM((SL,), jnp.float32),
            pltpu.VMEM((SL,), jnp.float32),
        ],
    )
    def scatter_kernel(mx_hbm, my_hbm, dst_hbm, zero_hbm, part_hbm, out_hbm,
                       acc_v, dst_v, msg_v, red_v, tmp_v):
        c = lax.axis_index("c")
        sid = lax.axis_index("s")
        pltpu.sync_copy(zero_hbm, acc_v)

        @pl.loop(0, nch)
        def _(ci):
            base = sid * shard + ci * C
            pltpu.sync_copy(dst_hbm.at[pl.ds(base, C)], dst_v)

            @pl.when(c == 0)
            def _():
                pltpu.sync_copy(mx_hbm.at[pl.ds(base, C)], msg_v)

            @pl.when(c == 1)
            def _():
                pltpu.sync_copy(my_hbm.at[pl.ds(base, C)], msg_v)

            @pl.loop(0, C // 16, unroll=8)
            def _(g):
                o = g * 16
                plsc.addupdate_scatter(
                    acc_v, [dst_v[pl.ds(o, 16)]], msg_v[pl.ds(o, 16)])

        wid = c * 16 + sid
        pltpu.sync_copy(acc_v, part_hbm.at[pl.ds(wid * NPAD, NPAD)])
        plsc.subcore_barrier()

        pltpu.sync_copy(part_hbm.at[pl.ds(c * 16 * NPAD + sid * SL, SL)],
                        red_v)

        @pl.loop(1, 16)
        def _(p):
            pltpu.sync_copy(
                part_hbm.at[pl.ds((c * 16 + p) * NPAD + sid * SL, SL)], tmp_v)

            @pl.loop(0, SL // 16, unroll=8)
            def _(g):
                o = g * 16
                red_v[pl.ds(o, 16)] = red_v[pl.ds(o, 16)] + tmp_v[pl.ds(o, 16)]

        pltpu.sync_copy(red_v, out_hbm.at[pl.ds(c * NPAD + sid * SL, SL)])

    return scatter_kernel, NPAD


def kernel(pos, edge_index, a, W1, b1, W2, b2, W3, b3):
    N = pos.shape[0]
    E = edge_index.shape[1]

    px = pos[:, 0]
    py = pos[:, 1]
    a0 = a[:, 0]
    a1 = a[:, 1]
    src = edge_index[0]
    dst = edge_index[1]

    w1p = jnp.concatenate(
        [W1.T, jnp.zeros((W1.shape[1], 3), jnp.float32)],
        axis=1).astype(jnp.bfloat16)
    b1c = b1.reshape(-1, 1)
    w2t = W2.T.astype(jnp.bfloat16)
    b2c = b2.reshape(-1, 1)
    w3p = jnp.concatenate(
        [W3.T, jnp.zeros((6, W3.shape[0]), jnp.float32)],
        axis=0).astype(jnp.bfloat16)
    b3c = jnp.concatenate([b3, jnp.zeros((6,), jnp.float32)]).reshape(-1, 1)

    dxs, dys, a0g, a1g = _make_gather(N, E)(px, py, a0, a1, src, dst)
    mx, my = _make_mlp(E)(dxs, dys, a0g, a1g, w1p, b1c, w2t, b2c, w3p, b3c)

    scatter, NPAD = _make_scatter(N, E)
    zeros_n = jnp.zeros((NPAD,), jnp.float32)
    _, accflat = scatter(mx, my, dst, zeros_n)
    return accflat.reshape(2, NPAD)[:, :N].T * YNORM


# speedup vs baseline: 58.0956x; 1.3406x over previous
import functools

import jax
import jax.numpy as jnp
from jax import lax
from jax.experimental import pallas as pl
from jax.experimental.pallas import tpu as pltpu
from jax.experimental.pallas import tpu_sc as plsc

MAX_R = 0.1
YNORM = 1.0

_SC_MESH = plsc.VectorSubcoreMesh(core_axis_name="c", subcore_axis_name="s")
_SC_PARAMS = pltpu.CompilerParams(needs_layout_passes=False)


def _make_gather(N, E):
    C = 8000
    layout = [(0, 10), (10, 10), (20, 5), (25, 5)]
    for _, cnt in layout:
        assert E % cnt == 0 and (E // cnt) % C == 0

    ev = jax.ShapeDtypeStruct((E,), jnp.float32)

    @functools.partial(
        pl.kernel,
        out_type=(ev, ev, ev, ev),
        mesh=_SC_MESH,
        compiler_params=_SC_PARAMS,
        scratch_types=[
            pltpu.VMEM((N,), jnp.float32),
            pltpu.VMEM((C,), jnp.int32),
            pltpu.VMEM((C,), jnp.int32),
            pltpu.VMEM((C,), jnp.float32),
        ],
    )
    def gather_kernel(px, py, a0h, a1h, src_hbm, dst_hbm,
                      odx, ody, oa0, oa1, tab_v, ids_v, idd_v, out_v):
        wid = lax.axis_index("c") * 16 + lax.axis_index("s")
        tabs = [px, py, a0h, a1h]
        outs = [odx, ody, oa0, oa1]

        for k, (start, cnt) in enumerate(layout):
            @pl.when((wid >= start) & (wid < start + cnt))
            def _(k=k, start=start, cnt=cnt):
                sid = wid - start
                shard = E // cnt
                nch = shard // C
                pltpu.sync_copy(tabs[k], tab_v)

                @pl.loop(0, nch)
                def _(ci):
                    base = sid * shard + ci * C
                    pltpu.sync_copy(dst_hbm.at[pl.ds(base, C)], idd_v)
                    if k < 2:
                        pltpu.sync_copy(src_hbm.at[pl.ds(base, C)], ids_v)

                        @plsc.parallel_loop(0, C, step=16, unroll=8)
                        def _(o):
                            vs = plsc.load_gather(tab_v, [ids_v[pl.ds(o, 16)]])
                            vd = plsc.load_gather(tab_v, [idd_v[pl.ds(o, 16)]])
                            out_v[pl.ds(o, 16)] = (vs - vd) * (1.0 / MAX_R)
                    else:
                        @plsc.parallel_loop(0, C, step=16, unroll=8)
                        def _(o):
                            out_v[pl.ds(o, 16)] = plsc.load_gather(
                                tab_v, [idd_v[pl.ds(o, 16)]])
                    pltpu.sync_copy(out_v, outs[k].at[pl.ds(base, C)])

    return gather_kernel


def _make_mlp(E):
    BLK = 5120
    Q = BLK // 4
    assert E % BLK == 0

    def mlp_body(dx_ref, dy_ref, a0_ref, a1_ref,
                 w1_ref, b1_ref, w2_ref, b2_ref, w3_ref, b3_ref,
                 ox_ref, oy_ref):
        dx = dx_ref[...]
        dy = dy_ref[...]
        a0 = a0_ref[...]
        a1 = a1_ref[...]
        zq = jnp.zeros((3, Q), jnp.float32)
        rows = []
        for k in range(4):
            d0 = dx[k * Q:(k + 1) * Q].reshape(1, Q)
            d1 = dy[k * Q:(k + 1) * Q].reshape(1, Q)
            r = jnp.sqrt(d0 * d0 + d1 * d1)
            rows += [d0, d1, r,
                     a0[k * Q:(k + 1) * Q].reshape(1, Q),
                     a1[k * Q:(k + 1) * Q].reshape(1, Q), zq]
        fp = jnp.concatenate(rows, axis=0)
        dn = (((1,), (0,)), ((), ()))
        h = lax.dot_general(w1_ref[...], fp.astype(jnp.bfloat16), dn,
                            preferred_element_type=jnp.float32) + b1_ref[...]
        h = jnp.maximum(h, 0.0).astype(jnp.bfloat16)
        h = lax.dot_general(w2_ref[...], h, dn,
                            preferred_element_type=jnp.float32) + b2_ref[...]
        h = jnp.maximum(h, 0.0).astype(jnp.bfloat16)
        o = lax.dot_general(w3_ref[...], h, dn,
                            preferred_element_type=jnp.float32) + b3_ref[...]
        ox_ref[...] = jnp.concatenate([o[0], o[2], o[4], o[6]], axis=0)
        oy_ref[...] = jnp.concatenate([o[1], o[3], o[5], o[7]], axis=0)

    ev = jax.ShapeDtypeStruct((E,), jnp.float32)
    edge_spec = pl.BlockSpec((BLK,), lambda i: (i,))
    full = lambda s: pl.BlockSpec(s, lambda i: (0, 0))
    return pl.pallas_call(
        mlp_body,
        grid=(E // BLK,),
        in_specs=[edge_spec, edge_spec, edge_spec, edge_spec,
                  full((256, 32)), full((256, 1)), full((256, 256)),
                  full((256, 1)), full((8, 256)), full((8, 1))],
        out_specs=(edge_spec, edge_spec),
        out_shape=(ev, ev),
    )


def _make_scatter(N, E):
    shard = E // 16
    C = 8000
    nch = shard // C
    assert shard % C == 0 and C % 16 == 0
    SL = -(-N // 16)
    SL = (SL + 7) // 8 * 8
    NPAD = SL * 16

    @functools.partial(
        pl.kernel,
        out_type=(jax.ShapeDtypeStruct((32 * NPAD,), jnp.float32),
                  jax.ShapeDtypeStruct((2 * NPAD,), jnp.float32)),
        mesh=_SC_MESH,
        compiler_params=_SC_PARAMS,
        scratch_types=[
            pltpu.VMEM((NPAD,), jnp.float32),
            pltpu.VMEM((C,), jnp.int32),
            pltpu.VMEM((C,), jnp.float32),
            pltpu.VMEM((SL,), jnp.float32),
            pltpu.VMEM((SL,), jnp.float32),
        ],
    )
    def scatter_kernel(mx_hbm, my_hbm, dst_hbm, zero_hbm, part_hbm, out_hbm,
                       acc_v, dst_v, msg_v, red_v, tmp_v):
        c = lax.axis_index("c")
        sid = lax.axis_index("s")
        pltpu.sync_copy(zero_hbm, acc_v)

        @pl.loop(0, nch)
        def _(ci):
            base = sid * shard + ci * C
            pltpu.sync_copy(dst_hbm.at[pl.ds(base, C)], dst_v)

            @pl.when(c == 0)
            def _():
                pltpu.sync_copy(mx_hbm.at[pl.ds(base, C)], msg_v)

            @pl.when(c == 1)
            def _():
                pltpu.sync_copy(my_hbm.at[pl.ds(base, C)], msg_v)

            @plsc.parallel_loop(0, C, step=16, unroll=8)
            def _(o):
                plsc.addupdate_scatter(
                    acc_v, [dst_v[pl.ds(o, 16)]], msg_v[pl.ds(o, 16)])

        wid = c * 16 + sid
        pltpu.sync_copy(acc_v, part_hbm.at[pl.ds(wid * NPAD, NPAD)])
        plsc.subcore_barrier()

        pltpu.sync_copy(part_hbm.at[pl.ds(c * 16 * NPAD + sid * SL, SL)],
                        red_v)

        @pl.loop(1, 16)
        def _(p):
            pltpu.sync_copy(
                part_hbm.at[pl.ds((c * 16 + p) * NPAD + sid * SL, SL)], tmp_v)

            @plsc.parallel_loop(0, SL, step=16, unroll=8)
            def _(o):
                red_v[pl.ds(o, 16)] = red_v[pl.ds(o, 16)] + tmp_v[pl.ds(o, 16)]

        pltpu.sync_copy(red_v, out_hbm.at[pl.ds(c * NPAD + sid * SL, SL)])

    return scatter_kernel, NPAD


def kernel(pos, edge_index, a, W1, b1, W2, b2, W3, b3):
    N = pos.shape[0]
    E = edge_index.shape[1]

    px = pos[:, 0]
    py = pos[:, 1]
    a0 = a[:, 0]
    a1 = a[:, 1]
    src = edge_index[0]
    dst = edge_index[1]

    eye4 = jnp.eye(4, dtype=jnp.float32)
    w1t8 = jnp.concatenate(
        [W1.T, jnp.zeros((W1.shape[1], 3), jnp.float32)], axis=1)
    w1bd = jnp.kron(eye4, w1t8).astype(jnp.bfloat16)
    b1bd = jnp.tile(b1, 4).reshape(-1, 1)
    w2bd = jnp.kron(eye4, W2.T).astype(jnp.bfloat16)
    b2bd = jnp.tile(b2, 4).reshape(-1, 1)
    w3bd = jnp.kron(eye4, W3.T).astype(jnp.bfloat16)
    b3bd = jnp.tile(b3, 4).reshape(-1, 1)

    dxs, dys, a0g, a1g = _make_gather(N, E)(px, py, a0, a1, src, dst)
    mx, my = _make_mlp(E)(dxs, dys, a0g, a1g,
                          w1bd, b1bd, w2bd, b2bd, w3bd, b3bd)

    scatter, NPAD = _make_scatter(N, E)
    zeros_n = jnp.zeros((NPAD,), jnp.float32)
    _, accflat = scatter(mx, my, dst, zeros_n)
    return accflat.reshape(2, NPAD)[:, :N].T * YNORM


# speedup vs baseline: 73.9865x; 1.2735x over previous
import functools

import jax
import jax.numpy as jnp
from jax import lax
from jax.experimental import pallas as pl
from jax.experimental.pallas import tpu as pltpu
from jax.experimental.pallas import tpu_sc as plsc

MAX_R = 0.1
YNORM = 1.0

_SC_MESH = plsc.VectorSubcoreMesh(core_axis_name="c", subcore_axis_name="s")
_SC_PARAMS = pltpu.CompilerParams(needs_layout_passes=False)


def _make_gather(N, E):
    C = 8000
    layout = [(0, 10), (10, 10), (20, 5), (25, 5)]
    for _, cnt in layout:
        assert E % cnt == 0 and (E // cnt) % C == 0

    ev = jax.ShapeDtypeStruct((E,), jnp.float32)

    @functools.partial(
        pl.kernel,
        out_type=(ev, ev, ev, ev),
        mesh=_SC_MESH,
        compiler_params=_SC_PARAMS,
        scratch_types=[
            pltpu.VMEM((N,), jnp.float32),
            pltpu.VMEM((C,), jnp.int32),
            pltpu.VMEM((C,), jnp.int32),
            pltpu.VMEM((C,), jnp.float32),
        ],
    )
    def gather_kernel(px, py, a0h, a1h, src_hbm, dst_hbm,
                      odx, ody, oa0, oa1, tab_v, ids_v, idd_v, out_v):
        wid = lax.axis_index("c") * 16 + lax.axis_index("s")
        tabs = [px, py, a0h, a1h]
        outs = [odx, ody, oa0, oa1]

        for k, (start, cnt) in enumerate(layout):
            @pl.when((wid >= start) & (wid < start + cnt))
            def _(k=k, start=start, cnt=cnt):
                sid = wid - start
                shard = E // cnt
                nch = shard // C
                pltpu.sync_copy(tabs[k], tab_v)

                @pl.loop(0, nch)
                def _(ci):
                    base = sid * shard + ci * C
                    pltpu.sync_copy(dst_hbm.at[pl.ds(base, C)], idd_v)
                    if k < 2:
                        pltpu.sync_copy(src_hbm.at[pl.ds(base, C)], ids_v)

                        @plsc.parallel_loop(0, C, step=16, unroll=8)
                        def _(o):
                            vs = plsc.load_gather(tab_v, [ids_v[pl.ds(o, 16)]])
                            vd = plsc.load_gather(tab_v, [idd_v[pl.ds(o, 16)]])
                            out_v[pl.ds(o, 16)] = (vs - vd) * (1.0 / MAX_R)
                    else:
                        @plsc.parallel_loop(0, C, step=16, unroll=8)
                        def _(o):
                            out_v[pl.ds(o, 16)] = plsc.load_gather(
                                tab_v, [idd_v[pl.ds(o, 16)]])
                    pltpu.sync_copy(out_v, outs[k].at[pl.ds(base, C)])

    return gather_kernel


def _make_mlp(E):
    BLK = 10240
    Q = BLK // 4
    assert E % BLK == 0

    def mlp_body(dx_ref, dy_ref, a0_ref, a1_ref,
                 w1_ref, w2_ref, b2_ref, w3_ref, b3_ref,
                 ox_ref, oy_ref):
        dx = dx_ref[...]
        dy = dy_ref[...]
        a0 = a0_ref[...]
        a1 = a1_ref[...]
        oq = jnp.ones((1, Q), jnp.float32)
        zq = jnp.zeros((2, Q), jnp.float32)
        rows = []
        for k in range(4):
            d0 = dx[k * Q:(k + 1) * Q].reshape(1, Q)
            d1 = dy[k * Q:(k + 1) * Q].reshape(1, Q)
            r = jnp.sqrt(d0 * d0 + d1 * d1)
            rows += [d0, d1, r,
                     a0[k * Q:(k + 1) * Q].reshape(1, Q),
                     a1[k * Q:(k + 1) * Q].reshape(1, Q), oq, zq]
        fp = jnp.concatenate(rows, axis=0)
        dn = (((1,), (0,)), ((), ()))
        h = lax.dot_general(w1_ref[...], fp.astype(jnp.bfloat16), dn,
                            preferred_element_type=jnp.float32)
        h = jnp.maximum(h.astype(jnp.bfloat16), 0)
        h = lax.dot_general(w2_ref[...], h, dn,
                            preferred_element_type=jnp.float32)
        h = jnp.maximum(h.astype(jnp.bfloat16) + b2_ref[...], 0)
        o = lax.dot_general(w3_ref[...], h, dn,
                            preferred_element_type=jnp.float32) + b3_ref[...]
        ox_ref[...] = jnp.concatenate([o[0], o[2], o[4], o[6]], axis=0)
        oy_ref[...] = jnp.concatenate([o[1], o[3], o[5], o[7]], axis=0)

    ev = jax.ShapeDtypeStruct((E,), jnp.float32)
    edge_spec = pl.BlockSpec((BLK,), lambda i: (i,))
    full = lambda s: pl.BlockSpec(s, lambda i: (0, 0))
    return pl.pallas_call(
        mlp_body,
        grid=(E // BLK,),
        in_specs=[edge_spec, edge_spec, edge_spec, edge_spec,
                  full((256, 32)), full((256, 256)),
                  full((256, 1)), full((8, 256)), full((8, 1))],
        out_specs=(edge_spec, edge_spec),
        out_shape=(ev, ev),
    )


def _make_scatter(N, E):
    shard = E // 16
    C = 8000
    nch = shard // C
    assert shard % C == 0 and C % 16 == 0
    SL = -(-N // 16)
    SL = (SL + 7) // 8 * 8
    NPAD = SL * 16

    @functools.partial(
        pl.kernel,
        out_type=(jax.ShapeDtypeStruct((32 * NPAD,), jnp.float32),
                  jax.ShapeDtypeStruct((2 * NPAD,), jnp.float32)),
        mesh=_SC_MESH,
        compiler_params=_SC_PARAMS,
        scratch_types=[
            pltpu.VMEM((NPAD,), jnp.float32),
            pltpu.VMEM((C,), jnp.int32),
            pltpu.VMEM((C,), jnp.float32),
            pltpu.VMEM((SL,), jnp.float32),
            pltpu.VMEM((SL,), jnp.float32),
        ],
    )
    def scatter_kernel(mx_hbm, my_hbm, dst_hbm, zero_hbm, part_hbm, out_hbm,
                       acc_v, dst_v, msg_v, red_v, tmp_v):
        c = lax.axis_index("c")
        sid = lax.axis_index("s")
        pltpu.sync_copy(zero_hbm, acc_v)

        @pl.loop(0, nch)
        def _(ci):
            base = sid * shard + ci * C
            pltpu.sync_copy(dst_hbm.at[pl.ds(base, C)], dst_v)

            @pl.when(c == 0)
            def _():
                pltpu.sync_copy(mx_hbm.at[pl.ds(base, C)], msg_v)

            @pl.when(c == 1)
            def _():
                pltpu.sync_copy(my_hbm.at[pl.ds(base, C)], msg_v)

            @plsc.parallel_loop(0, C, step=16, unroll=8)
            def _(o):
                plsc.addupdate_scatter(
                    acc_v, [dst_v[pl.ds(o, 16)]], msg_v[pl.ds(o, 16)])

        wid = c * 16 + sid
        pltpu.sync_copy(acc_v, part_hbm.at[pl.ds(wid * NPAD, NPAD)])
        plsc.subcore_barrier()

        pltpu.sync_copy(part_hbm.at[pl.ds(c * 16 * NPAD + sid * SL, SL)],
                        red_v)

        @pl.loop(1, 16)
        def _(p):
            pltpu.sync_copy(
                part_hbm.at[pl.ds((c * 16 + p) * NPAD + sid * SL, SL)], tmp_v)

            @plsc.parallel_loop(0, SL, step=16, unroll=8)
            def _(o):
                red_v[pl.ds(o, 16)] = red_v[pl.ds(o, 16)] + tmp_v[pl.ds(o, 16)]

        pltpu.sync_copy(red_v, out_hbm.at[pl.ds(c * NPAD + sid * SL, SL)])

    return scatter_kernel, NPAD


def kernel(pos, edge_index, a, W1, b1, W2, b2, W3, b3):
    N = pos.shape[0]
    E = edge_index.shape[1]

    px = pos[:, 0]
    py = pos[:, 1]
    a0 = a[:, 0]
    a1 = a[:, 1]
    src = edge_index[0]
    dst = edge_index[1]

    eye4 = jnp.eye(4, dtype=jnp.float32)
    w1t8 = jnp.concatenate(
        [W1.T, b1.reshape(-1, 1), jnp.zeros((W1.shape[1], 2), jnp.float32)],
        axis=1)
    w1bd = jnp.kron(eye4, w1t8).astype(jnp.bfloat16)
    w2bd = jnp.kron(eye4, W2.T).astype(jnp.bfloat16)
    b2bd = jnp.tile(b2, 4).reshape(-1, 1).astype(jnp.bfloat16)
    w3bd = jnp.kron(eye4, W3.T).astype(jnp.bfloat16)
    b3bd = jnp.tile(b3, 4).reshape(-1, 1)

    dxs, dys, a0g, a1g = _make_gather(N, E)(px, py, a0, a1, src, dst)
    mx, my = _make_mlp(E)(dxs, dys, a0g, a1g,
                          w1bd, w2bd, b2bd, w3bd, b3bd)

    scatter, NPAD = _make_scatter(N, E)
    zeros_n = jnp.zeros((NPAD,), jnp.float32)
    _, accflat = scatter(mx, my, dst, zeros_n)
    return accflat.reshape(2, NPAD)[:, :N].T * YNORM


# speedup vs baseline: 103.6304x; 1.4007x over previous
import functools

import jax
import jax.numpy as jnp
from jax import lax
from jax.experimental import pallas as pl
from jax.experimental.pallas import tpu as pltpu
from jax.experimental.pallas import tpu_sc as plsc

MAX_R = 0.1
YNORM = 1.0
P = 5

_SC_MESH = plsc.VectorSubcoreMesh(core_axis_name="c", subcore_axis_name="s")
_SC_PARAMS = pltpu.CompilerParams(needs_layout_passes=False)


def _make_gather(N, off, EC):
    C = 8000
    layout = [(0, 10), (10, 10), (20, 5), (25, 5)]
    for _, cnt in layout:
        assert EC % cnt == 0 and (EC // cnt) % C == 0

    ev = jax.ShapeDtypeStruct((EC,), jnp.float32)

    @functools.partial(
        pl.kernel,
        out_type=(ev, ev, ev, ev),
        mesh=_SC_MESH,
        compiler_params=_SC_PARAMS,
        scratch_types=[
            pltpu.VMEM((N,), jnp.float32),
            pltpu.VMEM((C,), jnp.int32),
            pltpu.VMEM((C,), jnp.int32),
            pltpu.VMEM((C,), jnp.float32),
        ],
    )
    def gather_kernel(px, py, a0h, a1h, src_hbm, dst_hbm,
                      odx, ody, oa0, oa1, tab_v, ids_v, idd_v, out_v):
        wid = lax.axis_index("c") * 16 + lax.axis_index("s")
        tabs = [px, py, a0h, a1h]
        outs = [odx, ody, oa0, oa1]

        for k, (start, cnt) in enumerate(layout):
            @pl.when((wid >= start) & (wid < start + cnt))
            def _(k=k, start=start, cnt=cnt):
                sid = wid - start
                shard = EC // cnt
                nch = shard // C
                pltpu.sync_copy(tabs[k], tab_v)

                @pl.loop(0, nch)
                def _(ci):
                    base = sid * shard + ci * C
                    pltpu.sync_copy(dst_hbm.at[pl.ds(off + base, C)], idd_v)
                    if k < 2:
                        pltpu.sync_copy(src_hbm.at[pl.ds(off + base, C)],
                                        ids_v)

                        @plsc.parallel_loop(0, C, step=16, unroll=8)
                        def _(o):
                            vs = plsc.load_gather(tab_v, [ids_v[pl.ds(o, 16)]])
                            vd = plsc.load_gather(tab_v, [idd_v[pl.ds(o, 16)]])
                            out_v[pl.ds(o, 16)] = (vs - vd) * (1.0 / MAX_R)
                    else:
                        @plsc.parallel_loop(0, C, step=16, unroll=8)
                        def _(o):
                            out_v[pl.ds(o, 16)] = plsc.load_gather(
                                tab_v, [idd_v[pl.ds(o, 16)]])
                    pltpu.sync_copy(out_v, outs[k].at[pl.ds(base, C)])

    return gather_kernel


def _make_mlp(EC):
    BLK = 10240
    Q = BLK // 4
    assert EC % BLK == 0

    def mlp_body(dx_ref, dy_ref, a0_ref, a1_ref,
                 w1_ref, w2_ref, b2_ref, w3_ref, b3_ref,
                 ox_ref, oy_ref):
        dx = dx_ref[...]
        dy = dy_ref[...]
        a0 = a0_ref[...]
        a1 = a1_ref[...]
        oq = jnp.ones((1, Q), jnp.float32)
        zq = jnp.zeros((2, Q), jnp.float32)
        rows = []
        for k in range(4):
            d0 = dx[k * Q:(k + 1) * Q].reshape(1, Q)
            d1 = dy[k * Q:(k + 1) * Q].reshape(1, Q)
            r = jnp.sqrt(d0 * d0 + d1 * d1)
            rows += [d0, d1, r,
                     a0[k * Q:(k + 1) * Q].reshape(1, Q),
                     a1[k * Q:(k + 1) * Q].reshape(1, Q), oq, zq]
        fp = jnp.concatenate(rows, axis=0)
        dn = (((1,), (0,)), ((), ()))
        h = lax.dot_general(w1_ref[...], fp.astype(jnp.bfloat16), dn,
                            preferred_element_type=jnp.float32)
        h = jnp.maximum(h.astype(jnp.bfloat16), 0)
        h = lax.dot_general(w2_ref[...], h, dn,
                            preferred_element_type=jnp.float32)
        h = jnp.maximum(h.astype(jnp.bfloat16) + b2_ref[...], 0)
        o = lax.dot_general(w3_ref[...], h, dn,
                            preferred_element_type=jnp.float32) + b3_ref[...]
        ox_ref[...] = jnp.concatenate([o[0], o[2], o[4], o[6]], axis=0)
        oy_ref[...] = jnp.concatenate([o[1], o[3], o[5], o[7]], axis=0)

    ev = jax.ShapeDtypeStruct((EC,), jnp.float32)
    edge_spec = pl.BlockSpec((BLK,), lambda i: (i,))
    full = lambda s: pl.BlockSpec(s, lambda i: (0, 0))
    return pl.pallas_call(
        mlp_body,
        grid=(EC // BLK,),
        in_specs=[edge_spec, edge_spec, edge_spec, edge_spec,
                  full((256, 32)), full((256, 256)),
                  full((256, 1)), full((8, 256)), full((8, 1))],
        out_specs=(edge_spec, edge_spec),
        out_shape=(ev, ev),
    )


def _make_scatter(N, off, EC, first, last):
    shard = EC // 16
    C = 8000
    nch = shard // C
    assert shard % C == 0 and C % 16 == 0
    SL = -(-N // 16)
    SL = (SL + 7) // 8 * 8
    NPAD = SL * 16

    parts = jax.ShapeDtypeStruct((32 * NPAD,), jnp.float32)
    out_type = (parts, jax.ShapeDtypeStruct((2 * NPAD,), jnp.float32)) \
        if last else (parts,)

    @functools.partial(
        pl.kernel,
        out_type=out_type,
        mesh=_SC_MESH,
        compiler_params=_SC_PARAMS,
        scratch_types=[
            pltpu.VMEM((NPAD,), jnp.float32),
            pltpu.VMEM((C,), jnp.int32),
            pltpu.VMEM((C,), jnp.float32),
            pltpu.VMEM((SL,), jnp.float32),
            pltpu.VMEM((SL,), jnp.float32),
        ],
    )
    def scatter_kernel(mx_hbm, my_hbm, dst_hbm, init_hbm, *refs):
        if last:
            part_hbm, out_hbm, acc_v, dst_v, msg_v, red_v, tmp_v = refs
        else:
            part_hbm, acc_v, dst_v, msg_v, red_v, tmp_v = refs
        c = lax.axis_index("c")
        sid = lax.axis_index("s")
        wid = c * 16 + sid
        if first:
            pltpu.sync_copy(init_hbm, acc_v)
        else:
            pltpu.sync_copy(init_hbm.at[pl.ds(wid * NPAD, NPAD)], acc_v)

        @pl.loop(0, nch)
        def _(ci):
            base = sid * shard + ci * C
            pltpu.sync_copy(dst_hbm.at[pl.ds(off + base, C)], dst_v)

            @pl.when(c == 0)
            def _():
                pltpu.sync_copy(mx_hbm.at[pl.ds(base, C)], msg_v)

            @pl.when(c == 1)
            def _():
                pltpu.sync_copy(my_hbm.at[pl.ds(base, C)], msg_v)

            @plsc.parallel_loop(0, C, step=16, unroll=8)
            def _(o):
                plsc.addupdate_scatter(
                    acc_v, [dst_v[pl.ds(o, 16)]], msg_v[pl.ds(o, 16)])

        pltpu.sync_copy(acc_v, part_hbm.at[pl.ds(wid * NPAD, NPAD)])
        if last:
            plsc.subcore_barrier()
            pltpu.sync_copy(
                part_hbm.at[pl.ds(c * 16 * NPAD + sid * SL, SL)], red_v)

            @pl.loop(1, 16)
            def _(p):
                pltpu.sync_copy(
                    part_hbm.at[pl.ds((c * 16 + p) * NPAD + sid * SL, SL)],
                    tmp_v)

                @plsc.parallel_loop(0, SL, step=16, unroll=8)
                def _(o):
                    red_v[pl.ds(o, 16)] = (red_v[pl.ds(o, 16)]
                                           + tmp_v[pl.ds(o, 16)])

            pltpu.sync_copy(red_v, out_hbm.at[pl.ds(c * NPAD + sid * SL, SL)])

    return scatter_kernel, NPAD


def kernel(pos, edge_index, a, W1, b1, W2, b2, W3, b3):
    N = pos.shape[0]
    E = edge_index.shape[1]
    EC = E // P

    px = pos[:, 0]
    py = pos[:, 1]
    a0 = a[:, 0]
    a1 = a[:, 1]
    src = edge_index[0]
    dst = edge_index[1]

    eye4 = jnp.eye(4, dtype=jnp.float32)
    w1t8 = jnp.concatenate(
        [W1.T, b1.reshape(-1, 1), jnp.zeros((W1.shape[1], 2), jnp.float32)],
        axis=1)
    w1bd = jnp.kron(eye4, w1t8).astype(jnp.bfloat16)
    w2bd = jnp.kron(eye4, W2.T).astype(jnp.bfloat16)
    b2bd = jnp.tile(b2, 4).reshape(-1, 1).astype(jnp.bfloat16)
    w3bd = jnp.kron(eye4, W3.T).astype(jnp.bfloat16)
    b3bd = jnp.tile(b3, 4).reshape(-1, 1)

    mlp = _make_mlp(EC)
    SL = ((-(-N // 16)) + 7) // 8 * 8
    NPAD = SL * 16
    zeros_n = jnp.zeros((NPAD,), jnp.float32)

    accflat = None
    prev = zeros_n
    for p in range(P):
        f4 = _make_gather(N, p * EC, EC)(px, py, a0, a1, src, dst)
        mx, my = mlp(*f4, w1bd, w2bd, b2bd, w3bd, b3bd)
        scatter, _ = _make_scatter(N, p * EC, EC, first=(p == 0),
                                   last=(p == P - 1))
        res = scatter(mx, my, dst, prev)
        prev = res[0]
        if p == P - 1:
            accflat = res[1]

    return accflat.reshape(2, NPAD)[:, :N].T * YNORM
